# baseline (speedup 1.0000x reference)
import functools

import jax
import jax.numpy as jnp
from jax import lax
from jax.experimental import pallas as pl
from jax.experimental.pallas import tpu as pltpu
from jax.experimental.pallas import tpu_sc as plsc

D_MODEL = 1024
D_HIDDEN = 4096
N_EXP = 8
SEQ = 2048
HB = 1024
NH = D_HIDDEN // HB
BT = 128
NBLK = 2 * SEQ // BT + N_EXP
PAD = NBLK * BT
NW = 32
TPW = SEQ // NW


def _prep_kernel(x_ref, wg_ref, bg_ref, w2_ref,
                 slota_ref, slotb_ref, g1_ref, g2_ref, eid_ref, w2s_ref):
    e = pl.program_id(0)
    h = pl.program_id(1)

    @pl.when((e == 0) & (h == 0))
    def _gate():
        logits = jnp.dot(x_ref[...], wg_ref[...],
                         preferred_element_type=jnp.float32) + bg_ref[...]
        col = lax.broadcasted_iota(jnp.int32, logits.shape, 1)
        v1 = jnp.max(logits, axis=1, keepdims=True)
        i1 = jnp.min(jnp.where(logits == v1, col, N_EXP), axis=1,
                     keepdims=True)
        masked = jnp.where(col == i1, -jnp.inf, logits)
        v2 = jnp.max(masked, axis=1, keepdims=True)
        i2 = jnp.min(jnp.where(masked == v2, col, N_EXP), axis=1,
                     keepdims=True)
        ev2 = jnp.exp(v2 - v1)
        g1_ref[...] = 1.0 / (1.0 + ev2)
        g2_ref[...] = ev2 / (1.0 + ev2)
        oh1 = (col == i1).astype(jnp.float32)
        oh2 = (col == i2).astype(jnp.float32)
        rr = lax.broadcasted_iota(jnp.int32, (SEQ, SEQ), 0)
        cc = lax.broadcasted_iota(jnp.int32, (SEQ, SEQ), 1)
        ltri = (rr >= cc).astype(jnp.float32)
        ca = jnp.dot(ltri, oh1, preferred_element_type=jnp.float32)
        cb = jnp.dot(ltri, oh2, preferred_element_type=jnp.float32)
        c1 = ca[SEQ - 1:SEQ, :]
        cnt = c1 + cb[SEQ - 1:SEQ, :]
        nblk_e = jnp.ceil(cnt / BT)
        re = lax.broadcasted_iota(jnp.int32, (N_EXP, N_EXP), 0)
        ce = lax.broadcasted_iota(jnp.int32, (N_EXP, N_EXP), 1)
        strict = (re < ce).astype(jnp.float32)
        po = jnp.dot(nblk_e, strict,
                     preferred_element_type=jnp.float32)
        ranka = jnp.sum(ca * oh1, axis=1, keepdims=True) - 1.0
        rankb = jnp.sum((cb + c1) * oh2, axis=1, keepdims=True) - 1.0
        offa = jnp.sum(po * oh1, axis=1, keepdims=True) * BT
        offb = jnp.sum(po * oh2, axis=1, keepdims=True) * BT
        slota_ref[...] = (offa + ranka).astype(jnp.int32)
        slotb_ref[...] = (offb + rankb).astype(jnp.int32)
        eye = (re == ce).astype(jnp.float32)
        po_col = jnp.sum(jnp.broadcast_to(po, (N_EXP, N_EXP)) * eye,
                         axis=1, keepdims=True)
        nb_col = jnp.sum(jnp.broadcast_to(nblk_e, (N_EXP, N_EXP)) * eye,
                         axis=1, keepdims=True)
        bi = lax.broadcasted_iota(jnp.int32, (N_EXP, 128), 1).astype(
            jnp.float32)
        valid = ((bi >= po_col) & (bi < po_col + nb_col)).astype(jnp.float32)
        ei = lax.broadcasted_iota(jnp.int32, (N_EXP, 128), 0).astype(
            jnp.float32)
        esum = jnp.sum(valid * ei, axis=0, keepdims=True)
        vany = jnp.sum(valid, axis=0, keepdims=True)
        eid_ref[...] = jnp.where(vany > 0.0, esum, -1.0).astype(jnp.int32)

    ones_col = jnp.ones((D_MODEL, 1), jnp.float32)
    w2s_ref[0] = jnp.dot(w2_ref[0], ones_col,
                         preferred_element_type=jnp.float32)


def _run_prep(xt, Wg, bg, W2):
    return pl.pallas_call(
        _prep_kernel,
        grid=(N_EXP, NH),
        in_specs=[
            pl.BlockSpec((SEQ, D_MODEL), lambda e, h: (0, 0)),
            pl.BlockSpec((D_MODEL, N_EXP), lambda e, h: (0, 0)),
            pl.BlockSpec((1, N_EXP), lambda e, h: (0, 0)),
            pl.BlockSpec((1, HB, D_MODEL), lambda e, h: (e, h, 0)),
        ],
        out_specs=[
            pl.BlockSpec((SEQ, 1), lambda e, h: (0, 0)),
            pl.BlockSpec((SEQ, 1), lambda e, h: (0, 0)),
            pl.BlockSpec((SEQ, 1), lambda e, h: (0, 0)),
            pl.BlockSpec((SEQ, 1), lambda e, h: (0, 0)),
            pl.BlockSpec((1, 128), lambda e, h: (0, 0)),
            pl.BlockSpec((1, HB, 1), lambda e, h: (e, h, 0)),
        ],
        out_shape=[
            jax.ShapeDtypeStruct((SEQ, 1), jnp.int32),
            jax.ShapeDtypeStruct((SEQ, 1), jnp.int32),
            jax.ShapeDtypeStruct((SEQ, 1), jnp.float32),
            jax.ShapeDtypeStruct((SEQ, 1), jnp.float32),
            jax.ShapeDtypeStruct((1, 128), jnp.int32),
            jax.ShapeDtypeStruct((N_EXP, D_HIDDEN, 1), jnp.float32),
        ],
    )(xt, Wg, bg.reshape(1, N_EXP), W2)


def _sc_scatter_body(x_hbm, slota_hbm, slotb_hbm, xg_hbm,
                     idxa_v, idxb_v, rows_v, sem):
    wid = lax.axis_index("s") * 2 + lax.axis_index("c")
    base = wid * TPW
    pltpu.sync_copy(slota_hbm.at[pl.ds(base, TPW)], idxa_v)
    pltpu.sync_copy(slotb_hbm.at[pl.ds(base, TPW)], idxb_v)
    pltpu.sync_copy(x_hbm.at[pl.ds(base, TPW)], rows_v)
    pltpu.async_copy(rows_v, xg_hbm.at[idxa_v], sem).wait()
    pltpu.async_copy(rows_v, xg_hbm.at[idxb_v], sem).wait()


def _run_sc_scatter(xt, slota, slotb):
    mesh = plsc.VectorSubcoreMesh(core_axis_name="c", subcore_axis_name="s")
    fn = functools.partial(
        pl.kernel, mesh=mesh,
        out_type=jax.ShapeDtypeStruct((PAD, D_MODEL), jnp.float32),
        scratch_types=[
            pltpu.VMEM((TPW,), jnp.int32),
            pltpu.VMEM((TPW,), jnp.int32),
            pltpu.VMEM((TPW, D_MODEL), jnp.float32),
            pltpu.SemaphoreType.DMA,
        ],
    )(_sc_scatter_body)
    return fn(xt, slota, slotb)


def _gemm_kernel(eid_ref, xg_ref, w1_ref, b1_ref, w2s_ref, b2_ref, out_ref):
    b = pl.program_id(0)
    h = pl.program_id(1)
    eid = eid_ref[b]

    @pl.when(h == 0)
    def _init():
        out_ref[...] = jnp.full((BT, 1), jnp.sum(b2_ref[...]), jnp.float32)

    @pl.when(eid >= 0)
    def _compute():
        pre = jnp.dot(xg_ref[...], w1_ref[0],
                      preferred_element_type=jnp.float32) + b1_ref[0]
        hact = jax.nn.gelu(pre, approximate=True)
        out_ref[...] += jnp.dot(hact, w2s_ref[0],
                                preferred_element_type=jnp.float32)


def _run_gemm(eid, xg, W1, b1, w2s, b2):
    def _e(b, h, eref):
        return jnp.maximum(eref[b], 0)

    return pl.pallas_call(
        _gemm_kernel,
        grid_spec=pltpu.PrefetchScalarGridSpec(
            num_scalar_prefetch=1,
            grid=(NBLK, NH),
            in_specs=[
                pl.BlockSpec((BT, D_MODEL), lambda b, h, eref: (b, 0)),
                pl.BlockSpec((1, D_MODEL, HB),
                             lambda b, h, eref: (_e(b, h, eref), 0, h)),
                pl.BlockSpec((1, 1, HB),
                             lambda b, h, eref: (_e(b, h, eref), 0, h)),
                pl.BlockSpec((1, HB, 1),
                             lambda b, h, eref: (_e(b, h, eref), h, 0)),
                pl.BlockSpec((1, 1, D_MODEL),
                             lambda b, h, eref: (_e(b, h, eref), 0, 0)),
            ],
            out_specs=pl.BlockSpec((BT, 1), lambda b, h, eref: (b, 0)),
        ),
        out_shape=jax.ShapeDtypeStruct((PAD, 1), jnp.float32),
    )(eid, xg, W1, b1.reshape(N_EXP, 1, D_HIDDEN), w2s,
      b2.reshape(N_EXP, 1, D_MODEL))


def _sc_combine_body(s_hbm, slota_hbm, slotb_hbm, g1_hbm, g2_hbm, out_hbm,
                     idxa_v, idxb_v, ga_v, gb_v, sa_v, sb_v, o_v, sem):
    wid = lax.axis_index("s") * 2 + lax.axis_index("c")
    base = wid * TPW
    pltpu.sync_copy(slota_hbm.at[pl.ds(base, TPW)], idxa_v)
    pltpu.sync_copy(slotb_hbm.at[pl.ds(base, TPW)], idxb_v)
    pltpu.sync_copy(g1_hbm.at[pl.ds(base, TPW)], ga_v)
    pltpu.sync_copy(g2_hbm.at[pl.ds(base, TPW)], gb_v)
    pltpu.async_copy(s_hbm.at[idxa_v], sa_v, sem).wait()
    pltpu.async_copy(s_hbm.at[idxb_v], sb_v, sem).wait()
    for j in range(TPW // 16):
        sl = pl.ds(j * 16, 16)
        o_v[sl] = ga_v[sl] * sa_v[sl] + gb_v[sl] * sb_v[sl]
    pltpu.sync_copy(o_v, out_hbm.at[pl.ds(base, TPW)])


def _run_sc_combine(s_sorted, slota, slotb, g1, g2):
    mesh = plsc.VectorSubcoreMesh(core_axis_name="c", subcore_axis_name="s")
    fn = functools.partial(
        pl.kernel, mesh=mesh,
        out_type=jax.ShapeDtypeStruct((SEQ,), jnp.float32),
        scratch_types=[
            pltpu.VMEM((TPW,), jnp.int32),
            pltpu.VMEM((TPW,), jnp.int32),
            pltpu.VMEM((TPW,), jnp.float32),
            pltpu.VMEM((TPW,), jnp.float32),
            pltpu.VMEM((TPW,), jnp.float32),
            pltpu.VMEM((TPW,), jnp.float32),
            pltpu.VMEM((TPW,), jnp.float32),
            pltpu.SemaphoreType.DMA,
        ],
    )(_sc_combine_body)
    return fn(s_sorted, slota, slotb, g1, g2)


def _lsm_kernel(s_ref, o_ref):
    sm = s_ref[...]
    m = jnp.max(sm)
    o_ref[...] = sm - m - jnp.log(jnp.sum(jnp.exp(sm - m)))


def _run_lsm(summed):
    return pl.pallas_call(
        _lsm_kernel,
        out_shape=jax.ShapeDtypeStruct((1, SEQ), jnp.float32),
    )(summed)


@jax.jit
def kernel(input, Wg, bg, W1, b1, W2, b2):
    B, S, D = input.shape
    xt = input.reshape(S, D)
    slota, slotb, g1, g2, eid, w2s = _run_prep(xt, Wg, bg, W2)
    slota = slota.reshape(SEQ)
    slotb = slotb.reshape(SEQ)
    xg = _run_sc_scatter(xt, slota, slotb)
    s_sorted = _run_gemm(eid.reshape(128)[:NBLK], xg, W1, b1, w2s, b2)
    summed = _run_sc_combine(s_sorted.reshape(PAD), slota, slotb,
                             g1.reshape(SEQ), g2.reshape(SEQ))
    return _run_lsm(summed.reshape(1, SEQ)).reshape(B, S)


# speedup vs baseline: 1.3557x; 1.3557x over previous
import functools

import jax
import jax.numpy as jnp
from jax import lax
from jax.experimental import pallas as pl
from jax.experimental.pallas import tpu as pltpu
from jax.experimental.pallas import tpu_sc as plsc

D_MODEL = 1024
D_HIDDEN = 4096
N_EXP = 8
SEQ = 2048
HB = 1024
NH = D_HIDDEN // HB
BT = 256
NBLK = 2 * SEQ // BT + N_EXP
PAD = NBLK * BT
NW = 32
TPW = SEQ // NW


def _prep_kernel(x_ref, wg_ref, bg_ref, w2_ref,
                 slota_ref, slotb_ref, g1_ref, g2_ref, eid_ref, w2s_ref):
    e = pl.program_id(0)
    h = pl.program_id(1)

    @pl.when((e == 0) & (h == 0))
    def _gate():
        logits = jnp.dot(x_ref[...], wg_ref[...],
                         preferred_element_type=jnp.float32) + bg_ref[...]
        col = lax.broadcasted_iota(jnp.int32, logits.shape, 1)
        v1 = jnp.max(logits, axis=1, keepdims=True)
        i1 = jnp.min(jnp.where(logits == v1, col, N_EXP), axis=1,
                     keepdims=True)
        masked = jnp.where(col == i1, -jnp.inf, logits)
        v2 = jnp.max(masked, axis=1, keepdims=True)
        i2 = jnp.min(jnp.where(masked == v2, col, N_EXP), axis=1,
                     keepdims=True)
        ev2 = jnp.exp(v2 - v1)
        g1_ref[...] = 1.0 / (1.0 + ev2)
        g2_ref[...] = ev2 / (1.0 + ev2)
        oh1 = (col == i1).astype(jnp.float32)
        oh2 = (col == i2).astype(jnp.float32)
        rr = lax.broadcasted_iota(jnp.int32, (SEQ, SEQ), 0)
        cc = lax.broadcasted_iota(jnp.int32, (SEQ, SEQ), 1)
        ltri = (rr >= cc).astype(jnp.float32)
        ca = jnp.dot(ltri, oh1, preferred_element_type=jnp.float32)
        cb = jnp.dot(ltri, oh2, preferred_element_type=jnp.float32)
        c1 = ca[SEQ - 1:SEQ, :]
        cnt = c1 + cb[SEQ - 1:SEQ, :]
        nblk_e = jnp.ceil(cnt / BT)
        re = lax.broadcasted_iota(jnp.int32, (N_EXP, N_EXP), 0)
        ce = lax.broadcasted_iota(jnp.int32, (N_EXP, N_EXP), 1)
        strict = (re < ce).astype(jnp.float32)
        po = jnp.dot(nblk_e, strict,
                     preferred_element_type=jnp.float32)
        ranka = jnp.sum(ca * oh1, axis=1, keepdims=True) - 1.0
        rankb = jnp.sum((cb + c1) * oh2, axis=1, keepdims=True) - 1.0
        offa = jnp.sum(po * oh1, axis=1, keepdims=True) * BT
        offb = jnp.sum(po * oh2, axis=1, keepdims=True) * BT
        slota_ref[...] = (offa + ranka).astype(jnp.int32)
        slotb_ref[...] = (offb + rankb).astype(jnp.int32)
        eye = (re == ce).astype(jnp.float32)
        po_col = jnp.sum(jnp.broadcast_to(po, (N_EXP, N_EXP)) * eye,
                         axis=1, keepdims=True)
        nb_col = jnp.sum(jnp.broadcast_to(nblk_e, (N_EXP, N_EXP)) * eye,
                         axis=1, keepdims=True)
        bi = lax.broadcasted_iota(jnp.int32, (N_EXP, 128), 1).astype(
            jnp.float32)
        valid = ((bi >= po_col) & (bi < po_col + nb_col)).astype(jnp.float32)
        ei = lax.broadcasted_iota(jnp.int32, (N_EXP, 128), 0).astype(
            jnp.float32)
        esum = jnp.sum(valid * ei, axis=0, keepdims=True)
        vany = jnp.sum(valid, axis=0, keepdims=True)
        eid_ref[...] = jnp.where(vany > 0.0, esum, -1.0).astype(jnp.int32)

    w2s_ref[0] = jnp.sum(w2_ref[0], axis=1, keepdims=True)


def _run_prep(xt, Wg, bg, W2):
    return pl.pallas_call(
        _prep_kernel,
        grid=(N_EXP, NH),
        in_specs=[
            pl.BlockSpec((SEQ, D_MODEL), lambda e, h: (0, 0)),
            pl.BlockSpec((D_MODEL, N_EXP), lambda e, h: (0, 0)),
            pl.BlockSpec((1, N_EXP), lambda e, h: (0, 0)),
            pl.BlockSpec((1, HB, D_MODEL), lambda e, h: (e, h, 0)),
        ],
        out_specs=[
            pl.BlockSpec((SEQ, 1), lambda e, h: (0, 0)),
            pl.BlockSpec((SEQ, 1), lambda e, h: (0, 0)),
            pl.BlockSpec((SEQ, 1), lambda e, h: (0, 0)),
            pl.BlockSpec((SEQ, 1), lambda e, h: (0, 0)),
            pl.BlockSpec((1, 128), lambda e, h: (0, 0)),
            pl.BlockSpec((1, HB, 1), lambda e, h: (e, h, 0)),
        ],
        out_shape=[
            jax.ShapeDtypeStruct((SEQ, 1), jnp.int32),
            jax.ShapeDtypeStruct((SEQ, 1), jnp.int32),
            jax.ShapeDtypeStruct((SEQ, 1), jnp.float32),
            jax.ShapeDtypeStruct((SEQ, 1), jnp.float32),
            jax.ShapeDtypeStruct((1, 128), jnp.int32),
            jax.ShapeDtypeStruct((N_EXP, D_HIDDEN, 1), jnp.float32),
        ],
    )(xt, Wg, bg.reshape(1, N_EXP), W2)


def _sc_scatter_body(x_hbm, slota_hbm, slotb_hbm, xg_hbm,
                     idxa_v, idxb_v, rows_v, sem):
    wid = lax.axis_index("s") * 2 + lax.axis_index("c")
    base = wid * TPW
    pltpu.sync_copy(slota_hbm.at[pl.ds(base, TPW)], idxa_v)
    pltpu.sync_copy(slotb_hbm.at[pl.ds(base, TPW)], idxb_v)
    pltpu.sync_copy(x_hbm.at[pl.ds(base, TPW)], rows_v)
    pltpu.async_copy(rows_v, xg_hbm.at[idxa_v], sem).wait()
    pltpu.async_copy(rows_v, xg_hbm.at[idxb_v], sem).wait()


def _run_sc_scatter(xt, slota, slotb):
    mesh = plsc.VectorSubcoreMesh(core_axis_name="c", subcore_axis_name="s")
    fn = functools.partial(
        pl.kernel, mesh=mesh,
        out_type=jax.ShapeDtypeStruct((PAD, D_MODEL), jnp.float32),
        scratch_types=[
            pltpu.VMEM((TPW,), jnp.int32),
            pltpu.VMEM((TPW,), jnp.int32),
            pltpu.VMEM((TPW, D_MODEL), jnp.float32),
            pltpu.SemaphoreType.DMA,
        ],
    )(_sc_scatter_body)
    return fn(xt, slota, slotb)


def _gemm_kernel(eid_ref, xg_ref, w1_ref, b1_ref, w2s_ref, b2_ref, out_ref):
    b = pl.program_id(0)
    h = pl.program_id(1)
    eid = eid_ref[b]

    @pl.when(h == 0)
    def _init():
        out_ref[...] = jnp.full((BT, 1), jnp.sum(b2_ref[...]), jnp.float32)

    @pl.when(eid >= 0)
    def _compute():
        pre = jnp.dot(xg_ref[...], w1_ref[0],
                      preferred_element_type=jnp.float32) + b1_ref[0]
        hact = jax.nn.gelu(pre, approximate=True)
        out_ref[...] += jnp.dot(hact, w2s_ref[0],
                                preferred_element_type=jnp.float32)


def _run_gemm(eid, xg, W1, b1, w2s, b2):
    def _e(b, h, eref):
        return jnp.maximum(eref[b], 0)

    return pl.pallas_call(
        _gemm_kernel,
        grid_spec=pltpu.PrefetchScalarGridSpec(
            num_scalar_prefetch=1,
            grid=(NBLK, NH),
            in_specs=[
                pl.BlockSpec((BT, D_MODEL), lambda b, h, eref: (b, 0)),
                pl.BlockSpec((1, D_MODEL, HB),
                             lambda b, h, eref: (_e(b, h, eref), 0, h)),
                pl.BlockSpec((1, 1, HB),
                             lambda b, h, eref: (_e(b, h, eref), 0, h)),
                pl.BlockSpec((1, HB, 1),
                             lambda b, h, eref: (_e(b, h, eref), h, 0)),
                pl.BlockSpec((1, 1, D_MODEL),
                             lambda b, h, eref: (_e(b, h, eref), 0, 0)),
            ],
            out_specs=pl.BlockSpec((BT, 1), lambda b, h, eref: (b, 0)),
        ),
        out_shape=jax.ShapeDtypeStruct((PAD, 1), jnp.float32),
    )(eid, xg, W1, b1.reshape(N_EXP, 1, D_HIDDEN), w2s,
      b2.reshape(N_EXP, 1, D_MODEL))


def _sc_combine_body(s_hbm, slota_hbm, slotb_hbm, g1_hbm, g2_hbm, out_hbm,
                     idxa_v, idxb_v, ga_v, gb_v, sa_v, sb_v, o_v, sem):
    wid = lax.axis_index("s") * 2 + lax.axis_index("c")
    base = wid * TPW
    pltpu.sync_copy(slota_hbm.at[pl.ds(base, TPW)], idxa_v)
    pltpu.sync_copy(slotb_hbm.at[pl.ds(base, TPW)], idxb_v)
    pltpu.sync_copy(g1_hbm.at[pl.ds(base, TPW)], ga_v)
    pltpu.sync_copy(g2_hbm.at[pl.ds(base, TPW)], gb_v)
    pltpu.async_copy(s_hbm.at[idxa_v], sa_v, sem).wait()
    pltpu.async_copy(s_hbm.at[idxb_v], sb_v, sem).wait()
    for j in range(TPW // 16):
        sl = pl.ds(j * 16, 16)
        o_v[sl] = ga_v[sl] * sa_v[sl] + gb_v[sl] * sb_v[sl]
    pltpu.sync_copy(o_v, out_hbm.at[pl.ds(base, TPW)])


def _run_sc_combine(s_sorted, slota, slotb, g1, g2):
    mesh = plsc.VectorSubcoreMesh(core_axis_name="c", subcore_axis_name="s")
    fn = functools.partial(
        pl.kernel, mesh=mesh,
        out_type=jax.ShapeDtypeStruct((SEQ,), jnp.float32),
        scratch_types=[
            pltpu.VMEM((TPW,), jnp.int32),
            pltpu.VMEM((TPW,), jnp.int32),
            pltpu.VMEM((TPW,), jnp.float32),
            pltpu.VMEM((TPW,), jnp.float32),
            pltpu.VMEM((TPW,), jnp.float32),
            pltpu.VMEM((TPW,), jnp.float32),
            pltpu.VMEM((TPW,), jnp.float32),
            pltpu.SemaphoreType.DMA,
        ],
    )(_sc_combine_body)
    return fn(s_sorted, slota, slotb, g1, g2)


def _lsm_kernel(s_ref, o_ref):
    sm = s_ref[...]
    m = jnp.max(sm)
    o_ref[...] = sm - m - jnp.log(jnp.sum(jnp.exp(sm - m)))


def _run_lsm(summed):
    return pl.pallas_call(
        _lsm_kernel,
        out_shape=jax.ShapeDtypeStruct((1, SEQ), jnp.float32),
    )(summed)


@jax.jit
def kernel(input, Wg, bg, W1, b1, W2, b2):
    B, S, D = input.shape
    xt = input.reshape(S, D)
    slota, slotb, g1, g2, eid, w2s = _run_prep(xt, Wg, bg, W2)
    slota = slota.reshape(SEQ)
    slotb = slotb.reshape(SEQ)
    xg = _run_sc_scatter(xt, slota, slotb)
    s_sorted = _run_gemm(eid.reshape(128)[:NBLK], xg, W1, b1, w2s, b2)
    summed = _run_sc_combine(s_sorted.reshape(PAD), slota, slotb,
                             g1.reshape(SEQ), g2.reshape(SEQ))
    return _run_lsm(summed.reshape(1, SEQ)).reshape(B, S)
